# baseline (speedup 1.0000x reference)
import jax
import jax.numpy as jnp
from jax.experimental import pallas as pl
from jax.experimental.pallas import tpu as pltpu


def _gcn_fused_kernel(feat_ref, adj_ref, mask_ref, w1t_ref, b1_ref,
                      w2t_ref, b2_ref, wfct_ref, bfc_ref, out_ref):
    X = feat_ref[0]
    A = adj_ref[0]
    m = mask_ref[0].reshape(-1, 1)

    A16 = A.astype(jnp.bfloat16)

    def split_hi_lo(p):
        hi = p.astype(jnp.bfloat16)
        lo = (p - hi.astype(jnp.float32)).astype(jnp.bfloat16)
        return jnp.concatenate([hi, lo], axis=1)

    p1 = jnp.dot(X, w1t_ref[...], preferred_element_type=jnp.float32)
    s1 = jnp.dot(A16, split_hi_lo(p1), preferred_element_type=jnp.float32)
    H1 = p1.shape[1]
    h1 = (s1[:, :H1] + s1[:, H1:] + b1_ref[...]) * m

    p2 = jnp.dot(h1, w2t_ref[...], preferred_element_type=jnp.float32)
    s2 = jnp.dot(A16, split_hi_lo(p2), preferred_element_type=jnp.float32)
    H2 = p2.shape[1]
    h2 = (s2[:, :H2] + s2[:, H2:] + b2_ref[...]) * m

    mx = jnp.max(h2, axis=0, keepdims=True)
    out_ref[0] = (jnp.dot(mx, wfct_ref[...],
                          preferred_element_type=jnp.float32) + bfc_ref[...])


def kernel(features, adj, mask, W1, b1, W2, b2, Wfc, bfc):
    B, N, F = features.shape
    H1 = W1.shape[0]
    H2 = W2.shape[0]
    OUT = Wfc.shape[0]

    w1t = W1.T
    w2t = W2.T
    wfct = Wfc.T
    b1r = b1.reshape(1, H1)
    b2r = b2.reshape(1, H2)
    bfcr = bfc.reshape(1, OUT)

    grid = (B,)
    out = pl.pallas_call(
        _gcn_fused_kernel,
        grid=grid,
        in_specs=[
            pl.BlockSpec((1, N, F), lambda b: (b, 0, 0)),
            pl.BlockSpec((1, N, N), lambda b: (b, 0, 0)),
            pl.BlockSpec((1, 1, N), lambda b: (b, 0, 0)),
            pl.BlockSpec((F, H1), lambda b: (0, 0)),
            pl.BlockSpec((1, H1), lambda b: (0, 0)),
            pl.BlockSpec((H1, H2), lambda b: (0, 0)),
            pl.BlockSpec((1, H2), lambda b: (0, 0)),
            pl.BlockSpec((H2, OUT), lambda b: (0, 0)),
            pl.BlockSpec((1, OUT), lambda b: (0, 0)),
        ],
        out_specs=pl.BlockSpec((1, 1, OUT), lambda b: (b, 0, 0)),
        out_shape=jax.ShapeDtypeStruct((B, 1, OUT), jnp.float32),
        compiler_params=pltpu.CompilerParams(
            dimension_semantics=("arbitrary",),
        ),
    )(features, adj, mask.reshape(B, 1, N), w1t, b1r, w2t, b2r, wfct, bfcr)
    return out.reshape(B, OUT)


# speedup vs baseline: 1.0537x; 1.0537x over previous
import jax
import jax.numpy as jnp
from jax.experimental import pallas as pl
from jax.experimental.pallas import tpu as pltpu


def _gcn_fused_kernel(feat_ref, adj_hbm, mask_ref, w1t_ref, b1_ref,
                      w2t_ref, b2_ref, wfct_ref, bfc_ref, out_ref,
                      abuf, sems):
    b = pl.program_id(0)
    nb = pl.num_programs(0)

    @pl.when(b == 0)
    def _start_first():
        pltpu.make_async_copy(adj_hbm.at[0], abuf.at[0], sems.at[0]).start()

    @pl.when(b + 1 < nb)
    def _prefetch_next():
        slot = (b + 1) % 2
        pltpu.make_async_copy(adj_hbm.at[b + 1], abuf.at[slot],
                              sems.at[slot]).start()

    cur = b % 2
    pltpu.make_async_copy(adj_hbm.at[b], abuf.at[cur], sems.at[cur]).wait()

    X = feat_ref[0]
    A = abuf[cur]
    m = mask_ref[0].reshape(-1, 1)

    p1 = jnp.dot(X, w1t_ref[...], preferred_element_type=jnp.float32)
    h1 = (jnp.dot(A, p1, preferred_element_type=jnp.float32) + b1_ref[...]) * m

    p2 = jnp.dot(h1, w2t_ref[...], preferred_element_type=jnp.float32)
    h2 = (jnp.dot(A, p2, preferred_element_type=jnp.float32) + b2_ref[...]) * m

    mx = jnp.max(h2, axis=0, keepdims=True)
    out_ref[0] = (jnp.dot(mx, wfct_ref[...],
                          preferred_element_type=jnp.float32) + bfc_ref[...])


def kernel(features, adj, mask, W1, b1, W2, b2, Wfc, bfc):
    B, N, F = features.shape
    H1 = W1.shape[0]
    H2 = W2.shape[0]
    OUT = Wfc.shape[0]

    w1t = W1.T
    w2t = W2.T
    wfct = Wfc.T
    b1r = b1.reshape(1, H1)
    b2r = b2.reshape(1, H2)
    bfcr = bfc.reshape(1, OUT)

    grid = (B,)
    out = pl.pallas_call(
        _gcn_fused_kernel,
        grid=grid,
        in_specs=[
            pl.BlockSpec((1, N, F), lambda b: (b, 0, 0)),
            pl.BlockSpec(memory_space=pl.ANY),
            pl.BlockSpec((1, 1, N), lambda b: (b, 0, 0)),
            pl.BlockSpec((F, H1), lambda b: (0, 0)),
            pl.BlockSpec((1, H1), lambda b: (0, 0)),
            pl.BlockSpec((H1, H2), lambda b: (0, 0)),
            pl.BlockSpec((1, H2), lambda b: (0, 0)),
            pl.BlockSpec((H2, OUT), lambda b: (0, 0)),
            pl.BlockSpec((1, OUT), lambda b: (0, 0)),
        ],
        out_specs=pl.BlockSpec((1, 1, OUT), lambda b: (b, 0, 0)),
        out_shape=jax.ShapeDtypeStruct((B, 1, OUT), jnp.float32),
        scratch_shapes=[
            pltpu.VMEM((2, N, N), jnp.float32),
            pltpu.SemaphoreType.DMA((2,)),
        ],
        compiler_params=pltpu.CompilerParams(
            dimension_semantics=("arbitrary",),
        ),
    )(features, adj, mask.reshape(B, 1, N), w1t, b1r, w2t, b2r, wfct, bfcr)
    return out.reshape(B, OUT)
